# baseline (speedup 1.0000x reference)
import functools

import jax
import jax.numpy as jnp
from jax import lax
from jax.experimental import pallas as pl
from jax.experimental.pallas import tpu as pltpu
from jax.experimental.pallas import tpu_sc as plsc

N_CLASS = 256
N_DOM = 3
N_SEG = N_CLASS * N_DOM
N_TOK = 16384
D = 512
NC, NS, L = 2, 16, 16
NW = NC * NS
NG = 8
NB = 4
CB = D // NB
TOK_G = N_TOK // NG
CHUNK = 32
NCHUNK = TOK_G // CHUNK
SEG_CHUNK = 512


def _sc_segment_sums(feature, label, domain_index, zrows):
    mesh = plsc.VectorSubcoreMesh(
        core_axis_name="c", subcore_axis_name="s", num_cores=NC, num_subcores=NS
    )

    @functools.partial(
        pl.kernel,
        out_type=(
            jax.ShapeDtypeStruct((NG, N_SEG, D), jnp.float32),
            jax.ShapeDtypeStruct((NW * N_SEG,), jnp.float32),
        ),
        mesh=mesh,
        compiler_params=pltpu.CompilerParams(needs_layout_passes=False),
        scratch_types=[
            pltpu.VMEM((N_SEG, CB), jnp.float32),
            pltpu.VMEM((2, CHUNK, CB), jnp.float32),
            pltpu.VMEM((TOK_G,), jnp.int32),
            pltpu.VMEM((SEG_CHUNK,), jnp.int32),
            pltpu.VMEM((SEG_CHUNK,), jnp.int32),
            pltpu.VMEM((L * N_SEG,), jnp.float32),
            pltpu.VMEM((N_SEG,), jnp.float32),
            pltpu.SemaphoreType.DMA,
            pltpu.SemaphoreType.DMA,
            pltpu.SemaphoreType.DMA,
        ],
    )
    def seg_sum_kernel(feature_hbm, label_hbm, domain_hbm, zrows_hbm,
                       sums_out, cnt_out,
                       acc, fbuf, seg_buf, lab, dom, cnt_lanes, cnt_tile,
                       sem0, sem1, semz):
        c = lax.axis_index("c")
        s = lax.axis_index("s")
        wid = c * NS + s
        g = wid // NB
        b = wid % NB
        tok0 = g * TOK_G
        col0 = b * CB
        z16 = jnp.zeros((L,), jnp.float32)
        o16 = jnp.ones((L,), jnp.float32)
        lanes = lax.iota(jnp.int32, L)

        zcp = pltpu.async_copy(zrows_hbm, acc, semz)

        def _feat_src(q):
            return feature_hbm.at[pl.ds(tok0 + q * CHUNK, CHUNK),
                                  pl.ds(col0, CB)]

        pltpu.async_copy(_feat_src(0), fbuf.at[0], sem0)
        pltpu.async_copy(_feat_src(1), fbuf.at[1], sem1)

        for i in range(L):
            for j in range(N_SEG // L):
                cnt_lanes[pl.ds(i * N_SEG + j * L, L)] = z16

        for p in range(TOK_G // SEG_CHUNK):
            pltpu.sync_copy(label_hbm.at[pl.ds(tok0 + p * SEG_CHUNK, SEG_CHUNK)], lab)
            pltpu.sync_copy(domain_hbm.at[pl.ds(tok0 + p * SEG_CHUNK, SEG_CHUNK)], dom)
            for j in range(SEG_CHUNK // L):
                seg = dom[pl.ds(j * L, L)] * N_CLASS + lab[pl.ds(j * L, L)]
                seg_buf[pl.ds(p * SEG_CHUNK + j * L, L)] = seg
                plsc.addupdate_scatter(cnt_lanes, [lanes * N_SEG + seg], o16)
        for j in range(N_SEG // L):
            acc16 = cnt_lanes[pl.ds(j * L, L)]
            for i in range(1, L):
                acc16 = acc16 + cnt_lanes[pl.ds(i * N_SEG + j * L, L)]
            cnt_tile[pl.ds(j * L, L)] = acc16
        pltpu.sync_copy(cnt_tile, cnt_out.at[pl.ds(wid * N_SEG, N_SEG)])

        def _accum_chunk(q, p):
            def _loads(i16, t):
                return [fbuf[p, i16 * L + t, pl.ds(h * L, L)]
                        for h in range(CB // L)]

            def _body(i16, _):
                seg16 = seg_buf[pl.ds(q * CHUNK + i16 * L, L)]
                vs = _loads(i16, 0)
                for t in range(L):
                    sg = seg16[t]
                    nxt = _loads(i16, t + 1) if t + 1 < L else None
                    for h in range(CB // L):
                        plsc.addupdate(acc.at[sg, pl.ds(h * L, L)], vs[h])
                    vs = nxt
                return 0
            lax.fori_loop(0, CHUNK // L, _body, 0)

        zcp.wait()

        def _pair(r, _):
            q0 = 2 * r
            pltpu.make_async_copy(_feat_src(q0), fbuf.at[0], sem0).wait()
            _accum_chunk(q0, 0)

            @pl.when(q0 + 2 < NCHUNK)
            def _():
                pltpu.async_copy(_feat_src(q0 + 2), fbuf.at[0], sem0)

            pltpu.make_async_copy(_feat_src(q0 + 1), fbuf.at[1], sem1).wait()
            _accum_chunk(q0 + 1, 1)

            @pl.when(q0 + 3 < NCHUNK)
            def _():
                pltpu.async_copy(_feat_src(q0 + 3), fbuf.at[1], sem1)
            return 0

        lax.fori_loop(0, NCHUNK // 2, _pair, 0)

        pltpu.sync_copy(acc, sums_out.at[g, :, pl.ds(col0, CB)])

    return seg_sum_kernel(feature, label, domain_index, zrows)


def _finalize(sums2, cnt2):
    def body(sums_ref, cnt_ref, out_ref):
        ssum = sums_ref[0]
        for g in range(1, NG):
            ssum = ssum + sums_ref[g]
        n = jnp.sum(cnt_ref[...], axis=0) * 0.25
        n = jnp.maximum(n, 1.0)
        cvec = ssum / n
        c1 = cvec[0:N_CLASS]
        c2 = cvec[N_CLASS:2 * N_CLASS]
        c3 = cvec[2 * N_CLASS:3 * N_CLASS]
        loss = (jnp.sum((c1 - c2) ** 2) + jnp.sum((c1 - c3) ** 2)
                + jnp.sum((c2 - c3) ** 2)) / jnp.float32(N_CLASS * D)
        out_ref[...] = jnp.full((1, 1), loss, jnp.float32)

    out = pl.pallas_call(
        body,
        out_shape=jax.ShapeDtypeStruct((1, 1), jnp.float32),
    )(sums2, cnt2)
    return out[0, 0]


def kernel(feature, label, domain_index):
    zrows = jnp.zeros((N_SEG, CB), jnp.float32)
    sums2, cnt_flat = _sc_segment_sums(
        feature, label.astype(jnp.int32), domain_index.astype(jnp.int32), zrows
    )
    return _finalize(sums2, cnt_flat.reshape(NW, N_SEG, 1))


# speedup vs baseline: 1.0340x; 1.0340x over previous
import functools

import jax
import jax.numpy as jnp
from jax import lax
from jax.experimental import pallas as pl
from jax.experimental.pallas import tpu as pltpu
from jax.experimental.pallas import tpu_sc as plsc

N_CLASS = 256
N_DOM = 3
N_SEG = N_CLASS * N_DOM
N_TOK = 16384
D = 512
NC, NS, L = 2, 16, 16
NW = NC * NS
NG = 8
NB = 4
CB = D // NB
TOK_G = N_TOK // NG
CHUNK = 32
NCHUNK = TOK_G // CHUNK
SEG_CHUNK = 512


def _sc_segment_sums(feature, label, domain_index, zrows):
    mesh = plsc.VectorSubcoreMesh(
        core_axis_name="c", subcore_axis_name="s", num_cores=NC, num_subcores=NS
    )

    @functools.partial(
        pl.kernel,
        out_type=(
            jax.ShapeDtypeStruct((NG, N_SEG, D), jnp.float32),
            jax.ShapeDtypeStruct((NW * N_SEG,), jnp.float32),
        ),
        mesh=mesh,
        compiler_params=pltpu.CompilerParams(needs_layout_passes=False),
        scratch_types=[
            pltpu.VMEM((N_SEG, CB), jnp.float32),
            pltpu.VMEM((2, CHUNK, CB), jnp.float32),
            pltpu.VMEM((TOK_G,), jnp.int32),
            pltpu.VMEM((SEG_CHUNK,), jnp.int32),
            pltpu.VMEM((SEG_CHUNK,), jnp.int32),
            pltpu.VMEM((L * N_SEG,), jnp.float32),
            pltpu.VMEM((N_SEG,), jnp.float32),
            pltpu.SemaphoreType.DMA,
            pltpu.SemaphoreType.DMA,
            pltpu.SemaphoreType.DMA,
        ],
    )
    def seg_sum_kernel(feature_hbm, label_hbm, domain_hbm, zrows_hbm,
                       sums_out, cnt_out,
                       acc, fbuf, seg_buf, lab, dom, cnt_lanes, cnt_tile,
                       sem0, sem1, semz):
        c = lax.axis_index("c")
        s = lax.axis_index("s")
        wid = c * NS + s
        g = wid // NB
        b = wid % NB
        tok0 = g * TOK_G
        col0 = b * CB
        z16 = jnp.zeros((L,), jnp.float32)
        o16 = jnp.ones((L,), jnp.float32)
        lanes = lax.iota(jnp.int32, L)

        def _feat_src(q):
            return feature_hbm.at[pl.ds(tok0 + q * CHUNK, CHUNK),
                                  pl.ds(col0, CB)]

        pltpu.async_copy(_feat_src(0), fbuf.at[0], sem0)
        pltpu.async_copy(_feat_src(1), fbuf.at[1], sem1)

        with jax.named_scope("zero_acc"):
            def _zero_row(i, _):
                for h in range(CB // L):
                    acc[i, pl.ds(h * L, L)] = z16
                return 0
            lax.fori_loop(0, N_SEG, _zero_row, 0)
        for i in range(L):
            for j in range(N_SEG // L):
                cnt_lanes[pl.ds(i * N_SEG + j * L, L)] = z16

        pl_scope = jax.named_scope("segcnt")
        pl_scope.__enter__()
        for p in range(TOK_G // SEG_CHUNK):
            pltpu.sync_copy(label_hbm.at[pl.ds(tok0 + p * SEG_CHUNK, SEG_CHUNK)], lab)
            pltpu.sync_copy(domain_hbm.at[pl.ds(tok0 + p * SEG_CHUNK, SEG_CHUNK)], dom)
            for j in range(SEG_CHUNK // L):
                seg = dom[pl.ds(j * L, L)] * N_CLASS + lab[pl.ds(j * L, L)]
                seg_buf[pl.ds(p * SEG_CHUNK + j * L, L)] = seg
                plsc.addupdate_scatter(cnt_lanes, [lanes * N_SEG + seg], o16)
        for j in range(N_SEG // L):
            acc16 = cnt_lanes[pl.ds(j * L, L)]
            for i in range(1, L):
                acc16 = acc16 + cnt_lanes[pl.ds(i * N_SEG + j * L, L)]
            cnt_tile[pl.ds(j * L, L)] = acc16
        pltpu.sync_copy(cnt_tile, cnt_out.at[pl.ds(wid * N_SEG, N_SEG)])
        pl_scope.__exit__(None, None, None)

        def _accum_chunk(q, p):
            def _loads(i16, t):
                return [fbuf[p, i16 * L + t, pl.ds(h * L, L)]
                        for h in range(CB // L)]

            def _body(i16, _):
                seg16 = seg_buf[pl.ds(q * CHUNK + i16 * L, L)]
                vs = _loads(i16, 0)
                for t in range(L):
                    sg = seg16[t]
                    nxt = _loads(i16, t + 1) if t + 1 < L else None
                    for h in range(CB // L):
                        plsc.addupdate(acc.at[sg, pl.ds(h * L, L)], vs[h])
                    vs = nxt
                return 0
            lax.fori_loop(0, CHUNK // L, _body, 0)

        mn_scope = jax.named_scope("main")
        mn_scope.__enter__()

        def _pair(r, _):
            q0 = 2 * r
            pltpu.make_async_copy(_feat_src(q0), fbuf.at[0], sem0).wait()
            _accum_chunk(q0, 0)

            @pl.when(q0 + 2 < NCHUNK)
            def _():
                pltpu.async_copy(_feat_src(q0 + 2), fbuf.at[0], sem0)

            pltpu.make_async_copy(_feat_src(q0 + 1), fbuf.at[1], sem1).wait()
            _accum_chunk(q0 + 1, 1)

            @pl.when(q0 + 3 < NCHUNK)
            def _():
                pltpu.async_copy(_feat_src(q0 + 3), fbuf.at[1], sem1)
            return 0

        lax.fori_loop(0, NCHUNK // 2, _pair, 0)
        mn_scope.__exit__(None, None, None)

        with jax.named_scope("publish"):
            pltpu.sync_copy(acc, sums_out.at[g, :, pl.ds(col0, CB)])

    return seg_sum_kernel(feature, label, domain_index, zrows)


def _finalize(sums2, cnt2):
    def body(sums_ref, cnt_ref, out_ref):
        ssum = sums_ref[0]
        for g in range(1, NG):
            ssum = ssum + sums_ref[g]
        n = jnp.sum(cnt_ref[...], axis=0) * 0.25
        n = jnp.maximum(n, 1.0)
        cvec = ssum / n
        c1 = cvec[0:N_CLASS]
        c2 = cvec[N_CLASS:2 * N_CLASS]
        c3 = cvec[2 * N_CLASS:3 * N_CLASS]
        loss = (jnp.sum((c1 - c2) ** 2) + jnp.sum((c1 - c3) ** 2)
                + jnp.sum((c2 - c3) ** 2)) / jnp.float32(N_CLASS * D)
        out_ref[...] = jnp.full((1, 1), loss, jnp.float32)

    out = pl.pallas_call(
        body,
        out_shape=jax.ShapeDtypeStruct((1, 1), jnp.float32),
    )(sums2, cnt2)
    return out[0, 0]


def kernel(feature, label, domain_index):
    zrows = jnp.zeros((N_SEG, CB), jnp.float32)
    sums2, cnt_flat = _sc_segment_sums(
        feature, label.astype(jnp.int32), domain_index.astype(jnp.int32), zrows
    )
    return _finalize(sums2, cnt_flat.reshape(NW, N_SEG, 1))


# speedup vs baseline: 1.3541x; 1.3095x over previous
import functools

import jax
import jax.numpy as jnp
from jax import lax
from jax.experimental import pallas as pl
from jax.experimental.pallas import tpu as pltpu
from jax.experimental.pallas import tpu_sc as plsc

N_CLASS = 256
N_DOM = 3
N_SEG = N_CLASS * N_DOM
N_TOK = 16384
D = 512
NC, NS, L = 2, 16, 16
NW = NC * NS
NG = 8
NB = 4
CB = D // NB
TOK_G = N_TOK // NG
CHUNK = 32
NCHUNK = TOK_G // CHUNK
SEG_CHUNK = 512


def _sc_segment_sums(feature, label, domain_index):
    mesh = plsc.VectorSubcoreMesh(
        core_axis_name="c", subcore_axis_name="s", num_cores=NC, num_subcores=NS
    )

    @functools.partial(
        pl.kernel,
        out_type=(
            jax.ShapeDtypeStruct((NG, N_SEG, D), jnp.float32),
            jax.ShapeDtypeStruct((NW, N_SEG), jnp.float32),
        ),
        mesh=mesh,
        compiler_params=pltpu.CompilerParams(needs_layout_passes=False),
        scratch_types=[
            pltpu.VMEM((N_SEG, CB), jnp.float32),
            pltpu.VMEM((2, CHUNK, CB), jnp.float32),
            pltpu.VMEM((TOK_G,), jnp.int32),
            pltpu.VMEM((TOK_G,), jnp.int32),
            pltpu.VMEM((L * N_SEG,), jnp.float32),
            pltpu.VMEM((N_SEG,), jnp.float32),
            pltpu.SemaphoreType.DMA,
            pltpu.SemaphoreType.DMA,
            pltpu.SemaphoreType.DMA,
            pltpu.SemaphoreType.DMA,
        ],
    )
    def seg_sum_kernel(feature_hbm, label_hbm, domain_hbm,
                       sums_out, cnt_out,
                       acc, fbuf, seg_buf, dom, cnt_lanes, cnt_tile,
                       sem0, sem1, seml, semd):
        c = lax.axis_index("c")
        s = lax.axis_index("s")
        wid = c * NS + s
        g = wid // NB
        b = wid % NB
        tok0 = g * TOK_G
        col0 = b * CB
        z16 = jnp.zeros((L,), jnp.float32)
        o16 = jnp.ones((L,), jnp.float32)
        lanes = lax.iota(jnp.int32, L)

        def _feat_src(q):
            return feature_hbm.at[pl.ds(tok0 + q * CHUNK, CHUNK),
                                  pl.ds(col0, CB)]

        pltpu.async_copy(_feat_src(0), fbuf.at[0], sem0)
        pltpu.async_copy(_feat_src(1), fbuf.at[1], sem1)
        lcp = pltpu.async_copy(label_hbm.at[pl.ds(tok0, TOK_G)], seg_buf, seml)
        dcp = pltpu.async_copy(domain_hbm.at[pl.ds(tok0, TOK_G)], dom, semd)

        with jax.named_scope("zero_acc"):
            def _zero_row(i, _):
                for h in range(CB // L):
                    acc[i, pl.ds(h * L, L)] = z16
                return 0
            lax.fori_loop(0, N_SEG, _zero_row, 0)
        for i in range(L):
            for j in range(N_SEG // L):
                cnt_lanes[pl.ds(i * N_SEG + j * L, L)] = z16

        pl_scope = jax.named_scope("segcnt")
        pl_scope.__enter__()
        lcp.wait()
        dcp.wait()
        for j in range(TOK_G // L):
            seg = dom[pl.ds(j * L, L)] * N_CLASS + seg_buf[pl.ds(j * L, L)]
            seg_buf[pl.ds(j * L, L)] = seg
            plsc.addupdate_scatter(cnt_lanes, [lanes * N_SEG + seg], o16)
        for j in range(N_SEG // L):
            acc16 = cnt_lanes[pl.ds(j * L, L)]
            for i in range(1, L):
                acc16 = acc16 + cnt_lanes[pl.ds(i * N_SEG + j * L, L)]
            cnt_tile[pl.ds(j * L, L)] = acc16
        pltpu.sync_copy(cnt_tile, cnt_out.at[wid])
        pl_scope.__exit__(None, None, None)

        def _accum_chunk(q, p):
            def _loads(i16, t):
                return [fbuf[p, i16 * L + t, pl.ds(h * L, L)]
                        for h in range(CB // L)]

            def _body(i16, _):
                seg16 = seg_buf[pl.ds(q * CHUNK + i16 * L, L)]
                vs = _loads(i16, 0)
                for t in range(L):
                    sg = seg16[t]
                    nxt = _loads(i16, t + 1) if t + 1 < L else None
                    for h in range(CB // L):
                        plsc.addupdate(acc.at[sg, pl.ds(h * L, L)], vs[h])
                    vs = nxt
                return 0
            lax.fori_loop(0, CHUNK // L, _body, 0)

        mn_scope = jax.named_scope("main")
        mn_scope.__enter__()

        def _pair(r, _):
            q0 = 2 * r
            pltpu.make_async_copy(_feat_src(q0), fbuf.at[0], sem0).wait()
            _accum_chunk(q0, 0)

            @pl.when(q0 + 2 < NCHUNK)
            def _():
                pltpu.async_copy(_feat_src(q0 + 2), fbuf.at[0], sem0)

            pltpu.make_async_copy(_feat_src(q0 + 1), fbuf.at[1], sem1).wait()
            _accum_chunk(q0 + 1, 1)

            @pl.when(q0 + 3 < NCHUNK)
            def _():
                pltpu.async_copy(_feat_src(q0 + 3), fbuf.at[1], sem1)
            return 0

        lax.fori_loop(0, NCHUNK // 2, _pair, 0)
        mn_scope.__exit__(None, None, None)

        with jax.named_scope("publish"):
            pltpu.sync_copy(acc, sums_out.at[g, :, pl.ds(col0, CB)])

    return seg_sum_kernel(feature, label, domain_index)


def _finalize(sums2, cnt2):
    def body(sums_ref, cnt_ref, out_ref):
        ssum = sums_ref[0]
        for g in range(1, NG):
            ssum = ssum + sums_ref[g]
        ones_col = jnp.full((NW, 1), 0.25, jnp.float32)
        n = lax.dot_general(cnt_ref[...], ones_col, (((0,), (0,)), ((), ())),
                            preferred_element_type=jnp.float32)
        n = jnp.maximum(n, 1.0)
        cvec = ssum / n
        c1 = cvec[0:N_CLASS]
        c2 = cvec[N_CLASS:2 * N_CLASS]
        c3 = cvec[2 * N_CLASS:3 * N_CLASS]
        loss = (jnp.sum((c1 - c2) ** 2) + jnp.sum((c1 - c3) ** 2)
                + jnp.sum((c2 - c3) ** 2)) / jnp.float32(N_CLASS * D)
        out_ref[...] = jnp.full((1, 1), loss, jnp.float32)

    out = pl.pallas_call(
        body,
        out_shape=jax.ShapeDtypeStruct((1, 1), jnp.float32),
    )(sums2, cnt2)
    return out[0, 0]


def kernel(feature, label, domain_index):
    sums2, cnt2 = _sc_segment_sums(
        feature, label.astype(jnp.int32), domain_index.astype(jnp.int32)
    )
    return _finalize(sums2, cnt2)
